# Initial kernel scaffold; baseline (speedup 1.0000x reference)
import jax
import jax.numpy as jnp
from jax.experimental import pallas as pl


def kernel(x, edge_index, W_src0, W_dst0, attn0, bias0, W_src1, W_dst1, attn1, bias1):
    raise NotImplementedError("write your pallas kernel here")


# speedup vs baseline: 5.1941x; 5.1941x over previous
import functools

import jax
import jax.numpy as jnp
from jax import lax
from jax.experimental import pallas as pl
from jax.experimental.pallas import tpu as pltpu
from jax.experimental.pallas import tpu_sc as plsc

N = 10000
E = 320000
DIN = 128
DHID = 128
DOUT = 64
NEG = 0.2

NC = 2
NS = 16
NW = NC * NS
EPW = E // NW
CH = 80
NCH = EPW // CH
NPAD = 10240
NPT = NPAD // NS
ROWS_PT = N // NS

_f32 = jnp.float32
_i32 = jnp.int32


def _mesh():
    return plsc.VectorSubcoreMesh(core_axis_name="c", subcore_axis_name="s")


def _iota16():
    return lax.iota(_i32, 16)


def _hsum16(buf, v):
    iota = _iota16()
    for k in (1, 2, 4, 8):
        buf[pl.ds(0, 16)] = v
        v = v + plsc.load_gather(buf, [jnp.bitwise_xor(iota, k)])
    return v


def _rmw_max(arr, d16, v16):
    def cond(pending):
        return jnp.any(pending)

    def body(pending):
        plsc.store_scatter(arr, [d16], v16, mask=pending)
        cur = plsc.load_gather(arr, [d16])
        return pending & (v16 > cur)
    lax.while_loop(cond, body, jnp.ones((16,), jnp.bool_))


def _rmw_add(arr, idarr, d16, v16):
    iota = _iota16()

    def cond(pending):
        return jnp.any(pending)

    def body(pending):
        plsc.store_scatter(idarr, [d16], iota, mask=pending)
        wid = plsc.load_gather(idarr, [d16])
        win = pending & (wid == iota)
        cur = plsc.load_gather(arr, [d16])
        plsc.store_scatter(arr, [d16], cur + v16, mask=win)
        return pending & jnp.logical_not(win)
    lax.while_loop(cond, body, jnp.ones((16,), jnp.bool_))


def _make_logits_max(d):
    nk = d // 16

    def body(fs_h, fd_h, src_h, dst_h, attn_h, log_h, maxp_h,
             srcb, dstb, fsr, fdr, logb, maxarr, comb, attnb, hbuf, stage):
        c = lax.axis_index("c")
        s = lax.axis_index("s")
        wid = c * NS + s
        ebase = wid * EPW
        pltpu.sync_copy(attn_h, attnb)
        av = [attnb[pl.ds(k * 16, 16)] for k in range(nk)]
        neginf = jnp.full((16,), -jnp.inf, _f32)

        def init(i, _):
            maxarr[pl.ds(i * 16, 16)] = neginf
            return 0
        lax.fori_loop(0, NPAD // 16, init, 0)

        def chunk(ci, _):
            off = ci * CH
            pltpu.sync_copy(src_h.at[pl.ds(ebase + off, CH)], srcb)
            pltpu.sync_copy(dst_h.at[pl.ds(ebase + off, CH)], dstb)
            pltpu.sync_copy(fs_h.at[srcb], fsr)
            pltpu.sync_copy(fd_h.at[dstb], fdr)

            iota = _iota16()

            def group(g, _):
                base = g * 16

                def edge(j, lvec):
                    i = base + j
                    acc = jnp.zeros((16,), _f32)
                    for k in range(nk):
                        v = fsr[i, pl.ds(k * 16, 16)] + fdr[i, pl.ds(k * 16, 16)]
                        acc = acc + jnp.maximum(v, NEG * v) * av[k]
                    l = _hsum16(hbuf, acc)
                    return jnp.where(iota == j, l, lvec)
                lvec = lax.fori_loop(0, 16, edge, jnp.zeros((16,), _f32))
                logb[pl.ds(off + base, 16)] = lvec
                _rmw_max(maxarr, dstb[pl.ds(base, 16)], lvec)
                return 0
            lax.fori_loop(0, CH // 16, group, 0)
            return 0
        lax.fori_loop(0, NCH, chunk, 0)
        pltpu.sync_copy(logb, log_h.at[pl.ds(ebase, EPW)])

        pltpu.sync_copy(maxarr, stage.at[pl.ds(s * NPAD, NPAD)])
        plsc.subcore_barrier()
        for t in range(NS):
            pltpu.sync_copy(stage.at[pl.ds(t * NPAD + s * NPT, NPT)],
                            comb.at[pl.ds(t * NPT, NPT)])

        def red(j, _):
            m = comb[pl.ds(j * 16, 16)]
            for t in range(1, NS):
                m = jnp.maximum(m, comb[pl.ds(t * NPT + j * 16, 16)])
            comb[pl.ds(j * 16, 16)] = m
            return 0
        lax.fori_loop(0, NPT // 16, red, 0)
        pltpu.sync_copy(comb.at[pl.ds(0, NPT)],
                        maxp_h.at[pl.ds(c * NPAD + s * NPT, NPT)])

    return pl.kernel(
        body,
        out_type=(jax.ShapeDtypeStruct((E,), _f32),
                  jax.ShapeDtypeStruct((NC * NPAD,), _f32)),
        mesh=_mesh(),
        compiler_params=pltpu.CompilerParams(needs_layout_passes=False, use_tc_tiling_on_sc=False),
        scratch_types=[
            pltpu.VMEM((CH,), _i32),
            pltpu.VMEM((CH,), _i32),
            pltpu.VMEM((CH, d), _f32),
            pltpu.VMEM((CH, d), _f32),
            pltpu.VMEM((EPW,), _f32),
            pltpu.VMEM((NPAD,), _f32),
            pltpu.VMEM((NS * NPT,), _f32),
            pltpu.VMEM((d,), _f32),
            pltpu.VMEM((128,), _f32),
            pltpu.VMEM_SHARED((NS * NPAD,), _f32),
        ],
    )


def _ee_denom_body(log_h, dst_h, maxp_h, ee_h, denp_h,
                   dstv, logv, eev, marr, mtmp, denarr, idarr, comb, stage):
    c = lax.axis_index("c")
    s = lax.axis_index("s")
    wid = c * NS + s
    ebase = wid * EPW
    pltpu.sync_copy(dst_h.at[pl.ds(ebase, EPW)], dstv)
    pltpu.sync_copy(log_h.at[pl.ds(ebase, EPW)], logv)
    pltpu.sync_copy(maxp_h.at[pl.ds(0, NPAD)], marr)
    pltpu.sync_copy(maxp_h.at[pl.ds(NPAD, NPAD)], mtmp)
    zero = jnp.zeros((16,), _f32)

    def initm(i, _):
        sl = pl.ds(i * 16, 16)
        marr[sl] = jnp.maximum(marr[sl], mtmp[sl])
        denarr[sl] = zero
        return 0
    lax.fori_loop(0, NPAD // 16, initm, 0)

    def vec(t, _):
        sl = pl.ds(t * 16, 16)
        d16 = dstv[sl]
        mg = plsc.load_gather(marr, [d16])
        eev[sl] = jnp.exp(logv[sl] - mg)
        return 0
    lax.fori_loop(0, EPW // 16, vec, 0)

    def acc(t, _):
        sl = pl.ds(t * 16, 16)
        _rmw_add(denarr, idarr, dstv[sl], eev[sl])
        return 0
    lax.fori_loop(0, EPW // 16, acc, 0)
    pltpu.sync_copy(eev, ee_h.at[pl.ds(ebase, EPW)])

    pltpu.sync_copy(denarr, stage.at[pl.ds(s * NPAD, NPAD)])
    plsc.subcore_barrier()
    for t in range(NS):
        pltpu.sync_copy(stage.at[pl.ds(t * NPAD + s * NPT, NPT)],
                        comb.at[pl.ds(t * NPT, NPT)])

    def red(j, _):
        m = comb[pl.ds(j * 16, 16)]
        for t in range(1, NS):
            m = m + comb[pl.ds(t * NPT + j * 16, 16)]
        comb[pl.ds(j * 16, 16)] = m
        return 0
    lax.fori_loop(0, NPT // 16, red, 0)
    pltpu.sync_copy(comb.at[pl.ds(0, NPT)],
                    denp_h.at[pl.ds(c * NPAD + s * NPT, NPT)])


def _make_ee_denom():
    return pl.kernel(
        _ee_denom_body,
        out_type=(jax.ShapeDtypeStruct((E,), _f32),
                  jax.ShapeDtypeStruct((NC * NPAD,), _f32)),
        mesh=_mesh(),
        compiler_params=pltpu.CompilerParams(needs_layout_passes=False, use_tc_tiling_on_sc=False),
        scratch_types=[
            pltpu.VMEM((EPW,), _i32),
            pltpu.VMEM((EPW,), _f32),
            pltpu.VMEM((EPW,), _f32),
            pltpu.VMEM((NPAD,), _f32),
            pltpu.VMEM((NPAD,), _f32),
            pltpu.VMEM((NPAD,), _f32),
            pltpu.VMEM((NPAD,), _i32),
            pltpu.VMEM((NS * NPT,), _f32),
            pltpu.VMEM_SHARED((NS * NPAD,), _f32),
        ],
    )


def _make_aggregate(d):
    nk = d // 16

    def body(fs_h, src_h, dst_h, ee_h, denp_h, parts_h,
             srcb, dstb, eeb, alphab, fsr, denarr, dtmp, zbuf, acc_s):
        c = lax.axis_index("c")
        s = lax.axis_index("s")
        wid = c * NS + s
        ebase = wid * EPW
        pltpu.sync_copy(denp_h.at[pl.ds(0, NPAD)], denarr)
        pltpu.sync_copy(denp_h.at[pl.ds(NPAD, NPAD)], dtmp)
        zero = jnp.zeros((16,), _f32)
        tiny = jnp.full((16,), 1e-9, _f32)

        def initd(i, _):
            sl = pl.ds(i * 16, 16)
            denarr[sl] = jnp.maximum(denarr[sl] + dtmp[sl], tiny)
            return 0
        lax.fori_loop(0, NPAD // 16, initd, 0)

        def initz(i, _):
            for k in range(nk):
                zbuf[i, pl.ds(k * 16, 16)] = zero
            return 0
        lax.fori_loop(0, 40, initz, 0)

        r0 = s * 640
        nz = jnp.where(s == NS - 1, 10, 16)

        def zacc(j, _):
            pltpu.sync_copy(zbuf, acc_s.at[pl.ds(r0 + j * 40, 40)])
            return 0
        lax.fori_loop(0, nz, zacc, 0)
        plsc.subcore_barrier()

        def chunk(ci, _):
            off = ci * CH
            pltpu.sync_copy(src_h.at[pl.ds(ebase + off, CH)], srcb)
            pltpu.sync_copy(dst_h.at[pl.ds(ebase + off, CH)], dstb)
            pltpu.sync_copy(ee_h.at[pl.ds(ebase + off, CH)], eeb)
            pltpu.sync_copy(fs_h.at[srcb], fsr)

            def avec(t, _):
                sl = pl.ds(t * 16, 16)
                d16 = dstb[sl]
                dv = plsc.load_gather(denarr, [d16])
                alphab[sl] = eeb[sl] / dv
                return 0
            lax.fori_loop(0, CH // 16, avec, 0)

            def scale(i, _):
                a = plsc.load_gather(alphab, [jnp.full((16,), 0, _i32) + i])
                for k in range(nk):
                    sl = pl.ds(k * 16, 16)
                    fsr[i, sl] = fsr[i, sl] * a
                return 0
            lax.fori_loop(0, CH, scale, 0)
            pltpu.sync_copy(fsr, acc_s.at[dstb], add=True)
            return 0
        lax.fori_loop(0, NCH, chunk, 0)
        plsc.subcore_barrier()

        @pl.when(s < NS - 1)
        def _():
            pltpu.sync_copy(acc_s.at[pl.ds(r0, 640)],
                            parts_h.at[c, pl.ds(r0, 640)])

        @pl.when(s == NS - 1)
        def _():
            pltpu.sync_copy(acc_s.at[pl.ds(r0, 400)],
                            parts_h.at[c, pl.ds(r0, 400)])

    return pl.kernel(
        body,
        out_type=jax.ShapeDtypeStruct((NC, N, d), _f32),
        mesh=_mesh(),
        compiler_params=pltpu.CompilerParams(needs_layout_passes=False, use_tc_tiling_on_sc=False),
        scratch_types=[
            pltpu.VMEM((CH,), _i32),
            pltpu.VMEM((CH,), _i32),
            pltpu.VMEM((CH,), _f32),
            pltpu.VMEM((128,), _f32),
            pltpu.VMEM((CH, d), _f32),
            pltpu.VMEM((NPAD,), _f32),
            pltpu.VMEM((NPAD,), _f32),
            pltpu.VMEM((40, d), _f32),
            pltpu.VMEM_SHARED((N, d), _f32),
        ],
    )


_RB = 80


def _mm2_body(x_r, wa_r, wb_r, oa_r, ob_r):
    x = x_r[...]
    oa_r[...] = jnp.dot(x, wa_r[...], preferred_element_type=_f32)
    ob_r[...] = jnp.dot(x, wb_r[...], preferred_element_type=_f32)


def _tc_matmul2(x, wa, wb):
    n, din = x.shape
    dout = wa.shape[1]
    return pl.pallas_call(
        _mm2_body,
        grid=(n // _RB,),
        in_specs=[
            pl.BlockSpec((_RB, din), lambda i: (i, 0)),
            pl.BlockSpec((din, dout), lambda i: (0, 0)),
            pl.BlockSpec((din, dout), lambda i: (0, 0)),
        ],
        out_specs=[
            pl.BlockSpec((_RB, dout), lambda i: (i, 0)),
            pl.BlockSpec((_RB, dout), lambda i: (i, 0)),
        ],
        out_shape=[
            jax.ShapeDtypeStruct((n, dout), _f32),
            jax.ShapeDtypeStruct((n, dout), _f32),
        ],
    )(x, wa, wb)


def _fuse_mm2_body(p0_r, p1_r, b_r, wa_r, wb_r, oa_r, ob_r):
    h = p0_r[...] + p1_r[...] + b_r[...]
    h = jnp.where(h > 0, h, jnp.exp(jnp.minimum(h, 0.0)) - 1.0)
    oa_r[...] = jnp.dot(h, wa_r[...], preferred_element_type=_f32)
    ob_r[...] = jnp.dot(h, wb_r[...], preferred_element_type=_f32)


def _tc_fuse_matmul2(p0, p1, bias, wa, wb):
    n, din = p0.shape
    dout = wa.shape[1]
    return pl.pallas_call(
        _fuse_mm2_body,
        grid=(n // _RB,),
        in_specs=[
            pl.BlockSpec((_RB, din), lambda i: (i, 0)),
            pl.BlockSpec((_RB, din), lambda i: (i, 0)),
            pl.BlockSpec((1, din), lambda i: (0, 0)),
            pl.BlockSpec((din, dout), lambda i: (0, 0)),
            pl.BlockSpec((din, dout), lambda i: (0, 0)),
        ],
        out_specs=[
            pl.BlockSpec((_RB, dout), lambda i: (i, 0)),
            pl.BlockSpec((_RB, dout), lambda i: (i, 0)),
        ],
        out_shape=[
            jax.ShapeDtypeStruct((n, dout), _f32),
            jax.ShapeDtypeStruct((n, dout), _f32),
        ],
    )(p0, p1, bias.reshape(1, din), wa, wb)


def _final_body(p0_r, p1_r, b_r, o_r):
    o_r[...] = p0_r[...] + p1_r[...] + b_r[...]


def _tc_final(p0, p1, bias):
    n, d = p0.shape
    return pl.pallas_call(
        _final_body,
        grid=(n // _RB,),
        in_specs=[
            pl.BlockSpec((_RB, d), lambda i: (i, 0)),
            pl.BlockSpec((_RB, d), lambda i: (i, 0)),
            pl.BlockSpec((1, d), lambda i: (0, 0)),
        ],
        out_specs=pl.BlockSpec((_RB, d), lambda i: (i, 0)),
        out_shape=jax.ShapeDtypeStruct((n, d), _f32),
    )(p0, p1, bias.reshape(1, d))


_logits_max_128 = _make_logits_max(DHID)
_logits_max_64 = _make_logits_max(DOUT)
_ee_denom = _make_ee_denom()
_aggregate_128 = _make_aggregate(DHID)
_aggregate_64 = _make_aggregate(DOUT)


def _edge_phase(fs, fd, src, dst, attn, d):
    lm = _logits_max_128 if d == DHID else _logits_max_64
    ag = _aggregate_128 if d == DHID else _aggregate_64
    logits, maxp = lm(fs, fd, src, dst, attn)
    ee, denp = _ee_denom(logits, dst, maxp)
    parts = ag(fs, src, dst, ee, denp)
    return parts[0], parts[1]


@jax.jit
def kernel(x, edge_index, W_src0, W_dst0, attn0, bias0,
           W_src1, W_dst1, attn1, bias1):
    src = edge_index[0]
    dst = edge_index[1]
    fs0, fd0 = _tc_matmul2(x, W_src0, W_dst0)
    p0a, p0b = _edge_phase(fs0, fd0, src, dst, attn0.reshape(-1), DHID)
    fs1, fd1 = _tc_fuse_matmul2(p0a, p0b, bias0, W_src1, W_dst1)
    p1a, p1b = _edge_phase(fs1, fd1, src, dst, attn1.reshape(-1), DOUT)
    return _tc_final(p1a, p1b, bias1)


# speedup vs baseline: 7.5770x; 1.4588x over previous
import functools

import jax
import jax.numpy as jnp
from jax import lax
from jax.experimental import pallas as pl
from jax.experimental.pallas import tpu as pltpu
from jax.experimental.pallas import tpu_sc as plsc

N = 10000
E = 320000
DIN = 128
DHID = 128
DOUT = 64
NEG = 0.2

NC = 2
NS = 16
NW = NC * NS
EPW = E // NW
CH = 80
NCH = EPW // CH
NPAD = 10240
NPT = NPAD // NS
ROWS_PT = N // NS

_f32 = jnp.float32
_i32 = jnp.int32


def _mesh():
    return plsc.VectorSubcoreMesh(core_axis_name="c", subcore_axis_name="s")


def _iota16():
    return lax.iota(_i32, 16)


def _rmw_max(arr, d16, v16):
    def cond(pending):
        return jnp.any(pending)

    def body(pending):
        plsc.store_scatter(arr, [d16], v16, mask=pending)
        cur = plsc.load_gather(arr, [d16])
        return pending & (v16 > cur)
    lax.while_loop(cond, body, jnp.ones((16,), jnp.bool_))


def _rmw_add(arr, idarr, d16, v16):
    iota = _iota16()

    def cond(pending):
        return jnp.any(pending)

    def body(pending):
        plsc.store_scatter(idarr, [d16], iota, mask=pending)
        wid = plsc.load_gather(idarr, [d16])
        win = pending & (wid == iota)
        cur = plsc.load_gather(arr, [d16])
        plsc.store_scatter(arr, [d16], cur + v16, mask=win)
        return pending & jnp.logical_not(win)
    lax.while_loop(cond, body, jnp.ones((16,), jnp.bool_))


def _make_logits_max(d):
    nk = d // 16

    def body(fs_h, fd_h, src_h, dst_h, attn_h, log_h, maxp_h,
             srcb0, srcb1, dstb0, dstb1, fsr0, fsr1, fdr0, fdr1,
             logb, maxarr, comb, attnb, mbuf, stage,
             semi0, semi1, semr0, semr1):
        c = lax.axis_index("c")
        s = lax.axis_index("s")
        ebase = (c * NS + s) * EPW
        pltpu.sync_copy(attn_h, attnb)
        av = [attnb[pl.ds(k * 16, 16)] for k in range(nk)]
        neginf = jnp.full((16,), -jnp.inf, _f32)

        def init(i, _):
            maxarr[pl.ds(i * 16, 16)] = neginf
            return 0
        lax.fori_loop(0, NPAD // 16, init, 0)

        iota = _iota16()
        colbase = iota * 16
        sb = (srcb0, srcb1)
        db = (dstb0, dstb1)
        fsb = (fsr0, fsr1)
        fdb = (fdr0, fdr1)
        si = (semi0, semi1)
        sr = (semr0, semr1)

        def issue_idx(g, b):
            pltpu.async_copy(src_h.at[pl.ds(ebase + g * CH, CH)], sb[b], si[b])
            pltpu.async_copy(dst_h.at[pl.ds(ebase + g * CH, CH)], db[b], si[b])

        def wait_idx(g, b):
            pltpu.make_async_copy(
                src_h.at[pl.ds(ebase + g * CH, CH)], sb[b], si[b]).wait()
            pltpu.make_async_copy(
                dst_h.at[pl.ds(ebase + g * CH, CH)], db[b], si[b]).wait()

        def issue_rows(b):
            pltpu.async_copy(fs_h.at[sb[b]], fsb[b], sr[b])
            pltpu.async_copy(fd_h.at[db[b]], fdb[b], sr[b])

        def wait_rows(b):
            pltpu.make_async_copy(fs_h.at[sb[b]], fsb[b], sr[b]).wait()
            pltpu.make_async_copy(fd_h.at[db[b]], fdb[b], sr[b]).wait()

        def compute(g, b):
            off = g * CH

            def group(gi, _):
                base = gi * 16
                for j in range(16):
                    row = base + j
                    acc = jnp.zeros((16,), _f32)
                    for k in range(nk):
                        sl = pl.ds(k * 16, 16)
                        v = fsb[b][row, sl] + fdb[b][row, sl]
                        acc = acc + jnp.maximum(v, NEG * v) * av[k]
                    mbuf[pl.ds(j * 16, 16)] = acc
                lvec = plsc.load_gather(mbuf, [colbase])
                for dd in range(1, 16):
                    lvec = lvec + plsc.load_gather(mbuf, [colbase + dd])
                logb[pl.ds(off + base, 16)] = lvec
                _rmw_max(maxarr, db[b][pl.ds(base, 16)], lvec)
                return 0
            lax.fori_loop(0, CH // 16, group, 0)

        issue_idx(0, 0)
        issue_idx(1, 1)
        wait_idx(0, 0)
        issue_rows(0)

        def outer(o, _):
            g0 = 2 * o

            @pl.when(g0 + 1 < NCH)
            def _():
                wait_idx(g0 + 1, 1)
                issue_rows(1)
            wait_rows(0)
            compute(g0, 0)

            @pl.when(g0 + 2 < NCH)
            def _():
                issue_idx(g0 + 2, 0)

            g1 = 2 * o + 1

            @pl.when(g1 < NCH)
            def _():
                @pl.when(g1 + 1 < NCH)
                def _():
                    wait_idx(g1 + 1, 0)
                    issue_rows(0)
                wait_rows(1)
                compute(g1, 1)

                @pl.when(g1 + 2 < NCH)
                def _():
                    issue_idx(g1 + 2, 1)
            return 0
        lax.fori_loop(0, (NCH + 1) // 2, outer, 0)
        pltpu.sync_copy(logb, log_h.at[pl.ds(ebase, EPW)])

        pltpu.sync_copy(maxarr, stage.at[pl.ds(s * NPAD, NPAD)])
        plsc.subcore_barrier()
        for t in range(NS):
            pltpu.sync_copy(stage.at[pl.ds(t * NPAD + s * NPT, NPT)],
                            comb.at[pl.ds(t * NPT, NPT)])

        def red(j, _):
            m = comb[pl.ds(j * 16, 16)]
            for t in range(1, NS):
                m = jnp.maximum(m, comb[pl.ds(t * NPT + j * 16, 16)])
            comb[pl.ds(j * 16, 16)] = m
            return 0
        lax.fori_loop(0, NPT // 16, red, 0)
        pltpu.sync_copy(comb.at[pl.ds(0, NPT)],
                        maxp_h.at[pl.ds(c * NPAD + s * NPT, NPT)])

    return pl.kernel(
        body,
        out_type=(jax.ShapeDtypeStruct((E,), _f32),
                  jax.ShapeDtypeStruct((NC * NPAD,), _f32)),
        mesh=_mesh(),
        compiler_params=pltpu.CompilerParams(needs_layout_passes=False, use_tc_tiling_on_sc=False),
        scratch_types=[
            pltpu.VMEM((CH,), _i32),
            pltpu.VMEM((CH,), _i32),
            pltpu.VMEM((CH,), _i32),
            pltpu.VMEM((CH,), _i32),
            pltpu.VMEM((CH, d), _f32),
            pltpu.VMEM((CH, d), _f32),
            pltpu.VMEM((CH, d), _f32),
            pltpu.VMEM((CH, d), _f32),
            pltpu.VMEM((EPW,), _f32),
            pltpu.VMEM((NPAD,), _f32),
            pltpu.VMEM((NS * NPT,), _f32),
            pltpu.VMEM((d,), _f32),
            pltpu.VMEM((256,), _f32),
            pltpu.VMEM_SHARED((NS * NPAD,), _f32),
            pltpu.SemaphoreType.DMA,
            pltpu.SemaphoreType.DMA,
            pltpu.SemaphoreType.DMA,
            pltpu.SemaphoreType.DMA,
        ],
    )


def _ee_denom_body(log_h, dst_h, maxp_h, ee_h, denp_h,
                   dstv, logv, eev, marr, mtmp, denarr, idarr, comb, stage):
    c = lax.axis_index("c")
    s = lax.axis_index("s")
    wid = c * NS + s
    ebase = wid * EPW
    pltpu.sync_copy(dst_h.at[pl.ds(ebase, EPW)], dstv)
    pltpu.sync_copy(log_h.at[pl.ds(ebase, EPW)], logv)
    pltpu.sync_copy(maxp_h.at[pl.ds(0, NPAD)], marr)
    pltpu.sync_copy(maxp_h.at[pl.ds(NPAD, NPAD)], mtmp)
    zero = jnp.zeros((16,), _f32)

    def initm(i, _):
        sl = pl.ds(i * 16, 16)
        marr[sl] = jnp.maximum(marr[sl], mtmp[sl])
        denarr[sl] = zero
        return 0
    lax.fori_loop(0, NPAD // 16, initm, 0)

    def vec(t, _):
        sl = pl.ds(t * 16, 16)
        d16 = dstv[sl]
        mg = plsc.load_gather(marr, [d16])
        eev[sl] = jnp.exp(logv[sl] - mg)
        return 0
    lax.fori_loop(0, EPW // 16, vec, 0)

    def acc(t, _):
        sl = pl.ds(t * 16, 16)
        _rmw_add(denarr, idarr, dstv[sl], eev[sl])
        return 0
    lax.fori_loop(0, EPW // 16, acc, 0)
    pltpu.sync_copy(eev, ee_h.at[pl.ds(ebase, EPW)])

    pltpu.sync_copy(denarr, stage.at[pl.ds(s * NPAD, NPAD)])
    plsc.subcore_barrier()
    for t in range(NS):
        pltpu.sync_copy(stage.at[pl.ds(t * NPAD + s * NPT, NPT)],
                        comb.at[pl.ds(t * NPT, NPT)])

    def red(j, _):
        m = comb[pl.ds(j * 16, 16)]
        for t in range(1, NS):
            m = m + comb[pl.ds(t * NPT + j * 16, 16)]
        comb[pl.ds(j * 16, 16)] = m
        return 0
    lax.fori_loop(0, NPT // 16, red, 0)
    pltpu.sync_copy(comb.at[pl.ds(0, NPT)],
                    denp_h.at[pl.ds(c * NPAD + s * NPT, NPT)])


def _make_ee_denom():
    return pl.kernel(
        _ee_denom_body,
        out_type=(jax.ShapeDtypeStruct((E,), _f32),
                  jax.ShapeDtypeStruct((NC * NPAD,), _f32)),
        mesh=_mesh(),
        compiler_params=pltpu.CompilerParams(needs_layout_passes=False, use_tc_tiling_on_sc=False),
        scratch_types=[
            pltpu.VMEM((EPW,), _i32),
            pltpu.VMEM((EPW,), _f32),
            pltpu.VMEM((EPW,), _f32),
            pltpu.VMEM((NPAD,), _f32),
            pltpu.VMEM((NPAD,), _f32),
            pltpu.VMEM((NPAD,), _f32),
            pltpu.VMEM((NPAD,), _i32),
            pltpu.VMEM((NS * NPT,), _f32),
            pltpu.VMEM_SHARED((NS * NPAD,), _f32),
        ],
    )


def _make_aggregate(d):
    nk = d // 16

    def body(fs_h, src_h, dst_h, ee_h, denp_h, parts_h,
             srcb, dstb, eeb, alphab, fsr, denarr, dtmp, zbuf, acc_s):
        c = lax.axis_index("c")
        s = lax.axis_index("s")
        wid = c * NS + s
        ebase = wid * EPW
        pltpu.sync_copy(denp_h.at[pl.ds(0, NPAD)], denarr)
        pltpu.sync_copy(denp_h.at[pl.ds(NPAD, NPAD)], dtmp)
        zero = jnp.zeros((16,), _f32)
        tiny = jnp.full((16,), 1e-9, _f32)

        def initd(i, _):
            sl = pl.ds(i * 16, 16)
            denarr[sl] = jnp.maximum(denarr[sl] + dtmp[sl], tiny)
            return 0
        lax.fori_loop(0, NPAD // 16, initd, 0)

        def initz(i, _):
            for k in range(nk):
                zbuf[i, pl.ds(k * 16, 16)] = zero
            return 0
        lax.fori_loop(0, 40, initz, 0)

        r0 = s * 640
        nz = jnp.where(s == NS - 1, 10, 16)

        def zacc(j, _):
            pltpu.sync_copy(zbuf, acc_s.at[pl.ds(r0 + j * 40, 40)])
            return 0
        lax.fori_loop(0, nz, zacc, 0)
        plsc.subcore_barrier()

        def chunk(ci, _):
            off = ci * CH
            pltpu.sync_copy(src_h.at[pl.ds(ebase + off, CH)], srcb)
            pltpu.sync_copy(dst_h.at[pl.ds(ebase + off, CH)], dstb)
            pltpu.sync_copy(ee_h.at[pl.ds(ebase + off, CH)], eeb)
            pltpu.sync_copy(fs_h.at[srcb], fsr)

            def avec(t, _):
                sl = pl.ds(t * 16, 16)
                d16 = dstb[sl]
                dv = plsc.load_gather(denarr, [d16])
                alphab[sl] = eeb[sl] / dv
                return 0
            lax.fori_loop(0, CH // 16, avec, 0)

            def scale(i, _):
                a = plsc.load_gather(alphab, [jnp.full((16,), 0, _i32) + i])
                for k in range(nk):
                    sl = pl.ds(k * 16, 16)
                    fsr[i, sl] = fsr[i, sl] * a
                return 0
            lax.fori_loop(0, CH, scale, 0)
            pltpu.sync_copy(fsr, acc_s.at[dstb], add=True)
            return 0
        lax.fori_loop(0, NCH, chunk, 0)
        plsc.subcore_barrier()

        @pl.when(s < NS - 1)
        def _():
            pltpu.sync_copy(acc_s.at[pl.ds(r0, 640)],
                            parts_h.at[c, pl.ds(r0, 640)])

        @pl.when(s == NS - 1)
        def _():
            pltpu.sync_copy(acc_s.at[pl.ds(r0, 400)],
                            parts_h.at[c, pl.ds(r0, 400)])

    return pl.kernel(
        body,
        out_type=jax.ShapeDtypeStruct((NC, N, d), _f32),
        mesh=_mesh(),
        compiler_params=pltpu.CompilerParams(needs_layout_passes=False, use_tc_tiling_on_sc=False),
        scratch_types=[
            pltpu.VMEM((CH,), _i32),
            pltpu.VMEM((CH,), _i32),
            pltpu.VMEM((CH,), _f32),
            pltpu.VMEM((128,), _f32),
            pltpu.VMEM((CH, d), _f32),
            pltpu.VMEM((NPAD,), _f32),
            pltpu.VMEM((NPAD,), _f32),
            pltpu.VMEM((40, d), _f32),
            pltpu.VMEM_SHARED((N, d), _f32),
        ],
    )


_RB = 80


def _mm2_body(x_r, wa_r, wb_r, oa_r, ob_r):
    x = x_r[...]
    oa_r[...] = jnp.dot(x, wa_r[...], preferred_element_type=_f32)
    ob_r[...] = jnp.dot(x, wb_r[...], preferred_element_type=_f32)


def _tc_matmul2(x, wa, wb):
    n, din = x.shape
    dout = wa.shape[1]
    return pl.pallas_call(
        _mm2_body,
        grid=(n // _RB,),
        in_specs=[
            pl.BlockSpec((_RB, din), lambda i: (i, 0)),
            pl.BlockSpec((din, dout), lambda i: (0, 0)),
            pl.BlockSpec((din, dout), lambda i: (0, 0)),
        ],
        out_specs=[
            pl.BlockSpec((_RB, dout), lambda i: (i, 0)),
            pl.BlockSpec((_RB, dout), lambda i: (i, 0)),
        ],
        out_shape=[
            jax.ShapeDtypeStruct((n, dout), _f32),
            jax.ShapeDtypeStruct((n, dout), _f32),
        ],
    )(x, wa, wb)


def _fuse_mm2_body(p0_r, p1_r, b_r, wa_r, wb_r, oa_r, ob_r):
    h = p0_r[...] + p1_r[...] + b_r[...]
    h = jnp.where(h > 0, h, jnp.exp(jnp.minimum(h, 0.0)) - 1.0)
    oa_r[...] = jnp.dot(h, wa_r[...], preferred_element_type=_f32)
    ob_r[...] = jnp.dot(h, wb_r[...], preferred_element_type=_f32)


def _tc_fuse_matmul2(p0, p1, bias, wa, wb):
    n, din = p0.shape
    dout = wa.shape[1]
    return pl.pallas_call(
        _fuse_mm2_body,
        grid=(n // _RB,),
        in_specs=[
            pl.BlockSpec((_RB, din), lambda i: (i, 0)),
            pl.BlockSpec((_RB, din), lambda i: (i, 0)),
            pl.BlockSpec((1, din), lambda i: (0, 0)),
            pl.BlockSpec((din, dout), lambda i: (0, 0)),
            pl.BlockSpec((din, dout), lambda i: (0, 0)),
        ],
        out_specs=[
            pl.BlockSpec((_RB, dout), lambda i: (i, 0)),
            pl.BlockSpec((_RB, dout), lambda i: (i, 0)),
        ],
        out_shape=[
            jax.ShapeDtypeStruct((n, dout), _f32),
            jax.ShapeDtypeStruct((n, dout), _f32),
        ],
    )(p0, p1, bias.reshape(1, din), wa, wb)


def _final_body(p0_r, p1_r, b_r, o_r):
    o_r[...] = p0_r[...] + p1_r[...] + b_r[...]


def _tc_final(p0, p1, bias):
    n, d = p0.shape
    return pl.pallas_call(
        _final_body,
        grid=(n // _RB,),
        in_specs=[
            pl.BlockSpec((_RB, d), lambda i: (i, 0)),
            pl.BlockSpec((_RB, d), lambda i: (i, 0)),
            pl.BlockSpec((1, d), lambda i: (0, 0)),
        ],
        out_specs=pl.BlockSpec((_RB, d), lambda i: (i, 0)),
        out_shape=jax.ShapeDtypeStruct((n, d), _f32),
    )(p0, p1, bias.reshape(1, d))


_logits_max_128 = _make_logits_max(DHID)
_logits_max_64 = _make_logits_max(DOUT)
_ee_denom = _make_ee_denom()
_aggregate_128 = _make_aggregate(DHID)
_aggregate_64 = _make_aggregate(DOUT)


def _edge_phase(fs, fd, src, dst, attn, d):
    lm = _logits_max_128 if d == DHID else _logits_max_64
    ag = _aggregate_128 if d == DHID else _aggregate_64
    logits, maxp = lm(fs, fd, src, dst, attn)
    ee, denp = _ee_denom(logits, dst, maxp)
    parts = ag(fs, src, dst, ee, denp)
    return parts[0], parts[1]


@jax.jit
def kernel(x, edge_index, W_src0, W_dst0, attn0, bias0,
           W_src1, W_dst1, attn1, bias1):
    src = edge_index[0]
    dst = edge_index[1]
    fs0, fd0 = _tc_matmul2(x, W_src0, W_dst0)
    p0a, p0b = _edge_phase(fs0, fd0, src, dst, attn0.reshape(-1), DHID)
    fs1, fd1 = _tc_fuse_matmul2(p0a, p0b, bias0, W_src1, W_dst1)
    p1a, p1b = _edge_phase(fs1, fd1, src, dst, attn1.reshape(-1), DOUT)
    return _tc_final(p1a, p1b, bias1)


# speedup vs baseline: 9.9041x; 1.3071x over previous
import functools

import jax
import jax.numpy as jnp
from jax import lax
from jax.experimental import pallas as pl
from jax.experimental.pallas import tpu as pltpu
from jax.experimental.pallas import tpu_sc as plsc

N = 10000
E = 320000
DIN = 128
DHID = 128
DOUT = 64
NEG = 0.2

NC = 2
NS = 16
NW = NC * NS
EPW = E // NW
CH = 80
NCH = EPW // CH
NPAD = 10240
NPT = NPAD // NS
ROWS_PT = N // NS

_f32 = jnp.float32
_i32 = jnp.int32


def _mesh():
    return plsc.VectorSubcoreMesh(core_axis_name="c", subcore_axis_name="s")


def _iota16():
    return lax.iota(_i32, 16)


def _rmw_max(arr, d16, v16):
    def cond(pending):
        return jnp.any(pending)

    def body(pending):
        plsc.store_scatter(arr, [d16], v16, mask=pending)
        cur = plsc.load_gather(arr, [d16])
        return pending & (v16 > cur)
    lax.while_loop(cond, body, jnp.ones((16,), jnp.bool_))


def _rmw_add(arr, idarr, d16, v16):
    iota = _iota16()

    def cond(pending):
        return jnp.any(pending)

    def body(pending):
        plsc.store_scatter(idarr, [d16], iota, mask=pending)
        wid = plsc.load_gather(idarr, [d16])
        win = pending & (wid == iota)
        cur = plsc.load_gather(arr, [d16])
        plsc.store_scatter(arr, [d16], cur + v16, mask=win)
        return pending & jnp.logical_not(win)
    lax.while_loop(cond, body, jnp.ones((16,), jnp.bool_))


def _make_logits_max(d):
    nk = d // 16

    def body(fs_h, fd_h, src_h, dst_h, attn_h, log_h, maxp_h,
             srcb0, srcb1, dstb0, dstb1, fsr0, fsr1, fdr0, fdr1,
             logb, maxarr, comb, attnb, mbuf, stage,
             semi0, semi1, semr0, semr1):
        c = lax.axis_index("c")
        s = lax.axis_index("s")
        ebase = (c * NS + s) * EPW
        pltpu.sync_copy(attn_h, attnb)
        av = [attnb[pl.ds(k * 16, 16)] for k in range(nk)]
        neginf = jnp.full((16,), -jnp.inf, _f32)

        def init(i, _):
            maxarr[pl.ds(i * 16, 16)] = neginf
            return 0
        lax.fori_loop(0, NPAD // 16, init, 0)

        iota = _iota16()
        colbase = iota * 16
        sb = (srcb0, srcb1)
        db = (dstb0, dstb1)
        fsb = (fsr0, fsr1)
        fdb = (fdr0, fdr1)
        si = (semi0, semi1)
        sr = (semr0, semr1)

        def issue_idx(g, b):
            pltpu.async_copy(src_h.at[pl.ds(ebase + g * CH, CH)], sb[b], si[b])
            pltpu.async_copy(dst_h.at[pl.ds(ebase + g * CH, CH)], db[b], si[b])

        def wait_idx(g, b):
            pltpu.make_async_copy(
                src_h.at[pl.ds(ebase + g * CH, CH)], sb[b], si[b]).wait()
            pltpu.make_async_copy(
                dst_h.at[pl.ds(ebase + g * CH, CH)], db[b], si[b]).wait()

        def issue_rows(b):
            pltpu.async_copy(fs_h.at[sb[b]], fsb[b], sr[b])
            pltpu.async_copy(fd_h.at[db[b]], fdb[b], sr[b])

        def wait_rows(b):
            pltpu.make_async_copy(fs_h.at[sb[b]], fsb[b], sr[b]).wait()
            pltpu.make_async_copy(fd_h.at[db[b]], fdb[b], sr[b]).wait()

        def compute(g, b):
            off = g * CH

            def group(gi, _):
                base = gi * 16
                for j in range(16):
                    row = base + j
                    acc = jnp.zeros((16,), _f32)
                    for k in range(nk):
                        sl = pl.ds(k * 16, 16)
                        v = fsb[b][row, sl] + fdb[b][row, sl]
                        acc = acc + jnp.maximum(v, NEG * v) * av[k]
                    mbuf[pl.ds(j * 16, 16)] = acc
                lvec = plsc.load_gather(mbuf, [colbase])
                for dd in range(1, 16):
                    lvec = lvec + plsc.load_gather(mbuf, [colbase + dd])
                logb[pl.ds(off + base, 16)] = lvec
                _rmw_max(maxarr, db[b][pl.ds(base, 16)], lvec)
                return 0
            lax.fori_loop(0, CH // 16, group, 0)

        issue_idx(0, 0)
        issue_idx(1, 1)
        wait_idx(0, 0)
        issue_rows(0)

        def outer(o, _):
            g0 = 2 * o

            @pl.when(g0 + 1 < NCH)
            def _():
                wait_idx(g0 + 1, 1)
                issue_rows(1)
            wait_rows(0)
            compute(g0, 0)

            @pl.when(g0 + 2 < NCH)
            def _():
                issue_idx(g0 + 2, 0)

            g1 = 2 * o + 1

            @pl.when(g1 < NCH)
            def _():
                @pl.when(g1 + 1 < NCH)
                def _():
                    wait_idx(g1 + 1, 0)
                    issue_rows(0)
                wait_rows(1)
                compute(g1, 1)

                @pl.when(g1 + 2 < NCH)
                def _():
                    issue_idx(g1 + 2, 1)
            return 0
        lax.fori_loop(0, (NCH + 1) // 2, outer, 0)
        pltpu.sync_copy(logb, log_h.at[pl.ds(ebase, EPW)])

        pltpu.sync_copy(maxarr, stage.at[pl.ds(s * NPAD, NPAD)])
        plsc.subcore_barrier()
        for t in range(NS):
            pltpu.sync_copy(stage.at[pl.ds(t * NPAD + s * NPT, NPT)],
                            comb.at[pl.ds(t * NPT, NPT)])

        def red(j, _):
            m = comb[pl.ds(j * 16, 16)]
            for t in range(1, NS):
                m = jnp.maximum(m, comb[pl.ds(t * NPT + j * 16, 16)])
            comb[pl.ds(j * 16, 16)] = m
            return 0
        lax.fori_loop(0, NPT // 16, red, 0)
        pltpu.sync_copy(comb.at[pl.ds(0, NPT)],
                        maxp_h.at[pl.ds(c * NPAD + s * NPT, NPT)])

    return pl.kernel(
        body,
        out_type=(jax.ShapeDtypeStruct((E,), _f32),
                  jax.ShapeDtypeStruct((NC * NPAD,), _f32)),
        mesh=_mesh(),
        compiler_params=pltpu.CompilerParams(needs_layout_passes=False, use_tc_tiling_on_sc=False),
        scratch_types=[
            pltpu.VMEM((CH,), _i32),
            pltpu.VMEM((CH,), _i32),
            pltpu.VMEM((CH,), _i32),
            pltpu.VMEM((CH,), _i32),
            pltpu.VMEM((CH, d), _f32),
            pltpu.VMEM((CH, d), _f32),
            pltpu.VMEM((CH, d), _f32),
            pltpu.VMEM((CH, d), _f32),
            pltpu.VMEM((EPW,), _f32),
            pltpu.VMEM((NPAD,), _f32),
            pltpu.VMEM((NS * NPT,), _f32),
            pltpu.VMEM((d,), _f32),
            pltpu.VMEM((256,), _f32),
            pltpu.VMEM_SHARED((NS * NPAD,), _f32),
            pltpu.SemaphoreType.DMA,
            pltpu.SemaphoreType.DMA,
            pltpu.SemaphoreType.DMA,
            pltpu.SemaphoreType.DMA,
        ],
    )


def _ee_denom_body(log_h, dst_h, maxp_h, ee_h, denp_h,
                   dstv, logv, eev, marr, mtmp, denarr, idarr, comb, stage):
    c = lax.axis_index("c")
    s = lax.axis_index("s")
    wid = c * NS + s
    ebase = wid * EPW
    pltpu.sync_copy(dst_h.at[pl.ds(ebase, EPW)], dstv)
    pltpu.sync_copy(log_h.at[pl.ds(ebase, EPW)], logv)
    pltpu.sync_copy(maxp_h.at[pl.ds(0, NPAD)], marr)
    pltpu.sync_copy(maxp_h.at[pl.ds(NPAD, NPAD)], mtmp)
    zero = jnp.zeros((16,), _f32)

    def initm(i, _):
        sl = pl.ds(i * 16, 16)
        marr[sl] = jnp.maximum(marr[sl], mtmp[sl])
        denarr[sl] = zero
        return 0
    lax.fori_loop(0, NPAD // 16, initm, 0)

    def vec(t, _):
        sl = pl.ds(t * 16, 16)
        d16 = dstv[sl]
        mg = plsc.load_gather(marr, [d16])
        eev[sl] = jnp.exp(logv[sl] - mg)
        return 0
    lax.fori_loop(0, EPW // 16, vec, 0)

    def acc(t, _):
        sl = pl.ds(t * 16, 16)
        _rmw_add(denarr, idarr, dstv[sl], eev[sl])
        return 0
    lax.fori_loop(0, EPW // 16, acc, 0)
    pltpu.sync_copy(eev, ee_h.at[pl.ds(ebase, EPW)])

    pltpu.sync_copy(denarr, stage.at[pl.ds(s * NPAD, NPAD)])
    plsc.subcore_barrier()
    for t in range(NS):
        pltpu.sync_copy(stage.at[pl.ds(t * NPAD + s * NPT, NPT)],
                        comb.at[pl.ds(t * NPT, NPT)])

    def red(j, _):
        m = comb[pl.ds(j * 16, 16)]
        for t in range(1, NS):
            m = m + comb[pl.ds(t * NPT + j * 16, 16)]
        comb[pl.ds(j * 16, 16)] = m
        return 0
    lax.fori_loop(0, NPT // 16, red, 0)
    pltpu.sync_copy(comb.at[pl.ds(0, NPT)],
                    denp_h.at[pl.ds(c * NPAD + s * NPT, NPT)])


def _make_ee_denom():
    return pl.kernel(
        _ee_denom_body,
        out_type=(jax.ShapeDtypeStruct((E,), _f32),
                  jax.ShapeDtypeStruct((NC * NPAD,), _f32)),
        mesh=_mesh(),
        compiler_params=pltpu.CompilerParams(needs_layout_passes=False, use_tc_tiling_on_sc=False),
        scratch_types=[
            pltpu.VMEM((EPW,), _i32),
            pltpu.VMEM((EPW,), _f32),
            pltpu.VMEM((EPW,), _f32),
            pltpu.VMEM((NPAD,), _f32),
            pltpu.VMEM((NPAD,), _f32),
            pltpu.VMEM((NPAD,), _f32),
            pltpu.VMEM((NPAD,), _i32),
            pltpu.VMEM((NS * NPT,), _f32),
            pltpu.VMEM_SHARED((NS * NPAD,), _f32),
        ],
    )


def _make_aggregate(d):
    nk = d // 16

    def body(fs_h, src_h, dst_h, ee_h, denp_h, parts_h,
             srcb0, srcb1, dstb0, dstb1, eeb0, eeb1, scb0, scb1,
             fsr0, fsr1, alphab, denarr, dtmp, zbuf, acc_s,
             semi0, semi1, semr0, semr1, sa0, sa1):
        c = lax.axis_index("c")
        s = lax.axis_index("s")
        ebase = (c * NS + s) * EPW
        pltpu.sync_copy(denp_h.at[pl.ds(0, NPAD)], denarr)
        pltpu.sync_copy(denp_h.at[pl.ds(NPAD, NPAD)], dtmp)
        zero = jnp.zeros((16,), _f32)
        tiny = jnp.full((16,), 1e-9, _f32)

        def initd(i, _):
            sl = pl.ds(i * 16, 16)
            denarr[sl] = jnp.maximum(denarr[sl] + dtmp[sl], tiny)
            return 0
        lax.fori_loop(0, NPAD // 16, initd, 0)

        def initz(i, _):
            for k in range(nk):
                zbuf[i, pl.ds(k * 16, 16)] = zero
            return 0
        lax.fori_loop(0, 40, initz, 0)

        r0 = s * 640
        nz = jnp.where(s == NS - 1, 10, 16)

        def zacc(j, _):
            pltpu.sync_copy(zbuf, acc_s.at[pl.ds(r0 + j * 40, 40)])
            return 0
        lax.fori_loop(0, nz, zacc, 0)
        plsc.subcore_barrier()

        sb = (srcb0, srcb1)
        db = (dstb0, dstb1)
        eb = (eeb0, eeb1)
        cb = (scb0, scb1)
        fsb = (fsr0, fsr1)
        si = (semi0, semi1)
        sr = (semr0, semr1)
        sa = (sa0, sa1)

        def issue_idx(g, b):
            sl = pl.ds(ebase + g * CH, CH)
            pltpu.async_copy(src_h.at[sl], sb[b], si[b])
            pltpu.async_copy(dst_h.at[sl], db[b], si[b])
            pltpu.async_copy(ee_h.at[sl], eb[b], si[b])

        def wait_idx(g, b):
            sl = pl.ds(ebase + g * CH, CH)
            pltpu.make_async_copy(src_h.at[sl], sb[b], si[b]).wait()
            pltpu.make_async_copy(dst_h.at[sl], db[b], si[b]).wait()
            pltpu.make_async_copy(ee_h.at[sl], eb[b], si[b]).wait()

        def issue_rows(b):
            pltpu.async_copy(fs_h.at[sb[b]], fsb[b], sr[b])

        def wait_rows(b):
            pltpu.make_async_copy(fs_h.at[sb[b]], fsb[b], sr[b]).wait()

        def issue_scatter(b):
            pltpu.async_copy(fsb[b], acc_s.at[cb[b]], sa[b], add=True)

        def wait_scatter(b):
            pltpu.make_async_copy(fsb[b], acc_s.at[cb[b]], sa[b]).wait()

        def compute(g, b):
            def group(gi, _):
                base = gi * 16
                sl16 = pl.ds(base, 16)
                d16 = db[b][sl16]
                dv = plsc.load_gather(denarr, [d16])
                alphab[sl16] = eb[b][sl16] / dv
                cb[b][sl16] = d16
                for j in range(16):
                    row = base + j
                    a = plsc.load_gather(
                        alphab, [jnp.full((16,), 0, _i32) + row])
                    for k in range(nk):
                        sl = pl.ds(k * 16, 16)
                        fsb[b][row, sl] = fsb[b][row, sl] * a
                return 0
            lax.fori_loop(0, CH // 16, group, 0)

        issue_idx(0, 0)
        issue_idx(1, 1)
        wait_idx(0, 0)
        issue_rows(0)

        def block(g, b):
            @pl.when(g + 1 < NCH)
            def _():
                @pl.when(g >= 1)
                def _():
                    wait_scatter(1 - b)
                wait_idx(g + 1, 1 - b)
                issue_rows(1 - b)
            wait_rows(b)
            compute(g, b)
            issue_scatter(b)

            @pl.when(g + 2 < NCH)
            def _():
                issue_idx(g + 2, b)

        def outer(o, _):
            block(2 * o, 0)

            @pl.when(2 * o + 1 < NCH)
            def _():
                block(2 * o + 1, 1)
            return 0
        lax.fori_loop(0, (NCH + 1) // 2, outer, 0)
        wait_scatter(0)
        wait_scatter(1)
        plsc.subcore_barrier()

        @pl.when(s < NS - 1)
        def _():
            pltpu.sync_copy(acc_s.at[pl.ds(r0, 640)],
                            parts_h.at[c, pl.ds(r0, 640)])

        @pl.when(s == NS - 1)
        def _():
            pltpu.sync_copy(acc_s.at[pl.ds(r0, 400)],
                            parts_h.at[c, pl.ds(r0, 400)])

    return pl.kernel(
        body,
        out_type=jax.ShapeDtypeStruct((NC, N, d), _f32),
        mesh=_mesh(),
        compiler_params=pltpu.CompilerParams(needs_layout_passes=False, use_tc_tiling_on_sc=False),
        scratch_types=[
            pltpu.VMEM((CH,), _i32),
            pltpu.VMEM((CH,), _i32),
            pltpu.VMEM((CH,), _i32),
            pltpu.VMEM((CH,), _i32),
            pltpu.VMEM((CH,), _f32),
            pltpu.VMEM((CH,), _f32),
            pltpu.VMEM((CH,), _i32),
            pltpu.VMEM((CH,), _i32),
            pltpu.VMEM((CH, d), _f32),
            pltpu.VMEM((CH, d), _f32),
            pltpu.VMEM((128,), _f32),
            pltpu.VMEM((NPAD,), _f32),
            pltpu.VMEM((NPAD,), _f32),
            pltpu.VMEM((40, d), _f32),
            pltpu.VMEM_SHARED((N, d), _f32),
            pltpu.SemaphoreType.DMA,
            pltpu.SemaphoreType.DMA,
            pltpu.SemaphoreType.DMA,
            pltpu.SemaphoreType.DMA,
            pltpu.SemaphoreType.DMA,
            pltpu.SemaphoreType.DMA,
        ],
    )


_RB = 80


def _mm2_body(x_r, wa_r, wb_r, oa_r, ob_r):
    x = x_r[...]
    oa_r[...] = jnp.dot(x, wa_r[...], preferred_element_type=_f32)
    ob_r[...] = jnp.dot(x, wb_r[...], preferred_element_type=_f32)


def _tc_matmul2(x, wa, wb):
    n, din = x.shape
    dout = wa.shape[1]
    return pl.pallas_call(
        _mm2_body,
        grid=(n // _RB,),
        in_specs=[
            pl.BlockSpec((_RB, din), lambda i: (i, 0)),
            pl.BlockSpec((din, dout), lambda i: (0, 0)),
            pl.BlockSpec((din, dout), lambda i: (0, 0)),
        ],
        out_specs=[
            pl.BlockSpec((_RB, dout), lambda i: (i, 0)),
            pl.BlockSpec((_RB, dout), lambda i: (i, 0)),
        ],
        out_shape=[
            jax.ShapeDtypeStruct((n, dout), _f32),
            jax.ShapeDtypeStruct((n, dout), _f32),
        ],
    )(x, wa, wb)


def _fuse_mm2_body(p0_r, p1_r, b_r, wa_r, wb_r, oa_r, ob_r):
    h = p0_r[...] + p1_r[...] + b_r[...]
    h = jnp.where(h > 0, h, jnp.exp(jnp.minimum(h, 0.0)) - 1.0)
    oa_r[...] = jnp.dot(h, wa_r[...], preferred_element_type=_f32)
    ob_r[...] = jnp.dot(h, wb_r[...], preferred_element_type=_f32)


def _tc_fuse_matmul2(p0, p1, bias, wa, wb):
    n, din = p0.shape
    dout = wa.shape[1]
    return pl.pallas_call(
        _fuse_mm2_body,
        grid=(n // _RB,),
        in_specs=[
            pl.BlockSpec((_RB, din), lambda i: (i, 0)),
            pl.BlockSpec((_RB, din), lambda i: (i, 0)),
            pl.BlockSpec((1, din), lambda i: (0, 0)),
            pl.BlockSpec((din, dout), lambda i: (0, 0)),
            pl.BlockSpec((din, dout), lambda i: (0, 0)),
        ],
        out_specs=[
            pl.BlockSpec((_RB, dout), lambda i: (i, 0)),
            pl.BlockSpec((_RB, dout), lambda i: (i, 0)),
        ],
        out_shape=[
            jax.ShapeDtypeStruct((n, dout), _f32),
            jax.ShapeDtypeStruct((n, dout), _f32),
        ],
    )(p0, p1, bias.reshape(1, din), wa, wb)


def _final_body(p0_r, p1_r, b_r, o_r):
    o_r[...] = p0_r[...] + p1_r[...] + b_r[...]


def _tc_final(p0, p1, bias):
    n, d = p0.shape
    return pl.pallas_call(
        _final_body,
        grid=(n // _RB,),
        in_specs=[
            pl.BlockSpec((_RB, d), lambda i: (i, 0)),
            pl.BlockSpec((_RB, d), lambda i: (i, 0)),
            pl.BlockSpec((1, d), lambda i: (0, 0)),
        ],
        out_specs=pl.BlockSpec((_RB, d), lambda i: (i, 0)),
        out_shape=jax.ShapeDtypeStruct((n, d), _f32),
    )(p0, p1, bias.reshape(1, d))


_logits_max_128 = _make_logits_max(DHID)
_logits_max_64 = _make_logits_max(DOUT)
_ee_denom = _make_ee_denom()
_aggregate_128 = _make_aggregate(DHID)
_aggregate_64 = _make_aggregate(DOUT)


def _edge_phase(fs, fd, src, dst, attn, d):
    lm = _logits_max_128 if d == DHID else _logits_max_64
    ag = _aggregate_128 if d == DHID else _aggregate_64
    logits, maxp = lm(fs, fd, src, dst, attn)
    ee, denp = _ee_denom(logits, dst, maxp)
    parts = ag(fs, src, dst, ee, denp)
    return parts[0], parts[1]


@jax.jit
def kernel(x, edge_index, W_src0, W_dst0, attn0, bias0,
           W_src1, W_dst1, attn1, bias1):
    src = edge_index[0]
    dst = edge_index[1]
    fs0, fd0 = _tc_matmul2(x, W_src0, W_dst0)
    p0a, p0b = _edge_phase(fs0, fd0, src, dst, attn0.reshape(-1), DHID)
    fs1, fd1 = _tc_fuse_matmul2(p0a, p0b, bias0, W_src1, W_dst1)
    p1a, p1b = _edge_phase(fs1, fd1, src, dst, attn1.reshape(-1), DOUT)
    return _tc_final(p1a, p1b, bias1)
